# baseline (device time: 135628 ns/iter reference)
import jax
import jax.numpy as jnp
from jax import lax
from jax.experimental import pallas as pl
from jax.experimental.pallas import tpu as pltpu

N_DEV = 8
MASKS = (1, 3, 4)
N_LAYERS = 3
N_EXCH = N_LAYERS * len(MASKS)


def kernel(x, Win0, Wout0, Win1, Wout1, Win2, Wout2):
    b, d_shard = x.shape
    h_dim = Win0.shape[1]

    def body(x_ref, win0_ref, wout0_ref, win1_ref, wout1_ref, win2_ref,
             wout2_ref, out_ref, acc_ref, recv_ref, x_cur_ref,
             send_sems, recv_sems):
        my = lax.axis_index("i")

        x_cur_ref[...] = x_ref[...]
        wins = (win0_ref, win1_ref, win2_ref)
        wouts = (wout0_ref, wout1_ref, wout2_ref)

        for l in range(N_LAYERS):
            acc_ref[...] = jnp.dot(
                x_cur_ref[...], wins[l][...],
                preferred_element_type=jnp.float32,
            )
            for r, mask in enumerate(MASKS):
                e = l * len(MASKS) + r
                partner = my ^ mask
                rdma = pltpu.make_async_remote_copy(
                    src_ref=acc_ref,
                    dst_ref=recv_ref.at[e],
                    send_sem=send_sems.at[e],
                    recv_sem=recv_sems.at[e],
                    device_id=(partner,),
                    device_id_type=pl.DeviceIdType.MESH,
                )
                rdma.start()
                rdma.wait()
                acc_ref[...] = acc_ref[...] + recv_ref[e]
            h = jnp.maximum(acc_ref[...], 0.0)
            x_cur_ref[...] = jnp.dot(
                h, wouts[l][...], preferred_element_type=jnp.float32,
            )
        out_ref[...] = x_cur_ref[...]

    return pl.pallas_call(
        body,
        out_shape=jax.ShapeDtypeStruct((b, d_shard), jnp.float32),
        in_specs=[pl.BlockSpec(memory_space=pltpu.VMEM)] * 7,
        out_specs=pl.BlockSpec(memory_space=pltpu.VMEM),
        scratch_shapes=[
            pltpu.VMEM((b, h_dim), jnp.float32),
            pltpu.VMEM((N_EXCH, b, h_dim), jnp.float32),
            pltpu.VMEM((b, d_shard), jnp.float32),
            pltpu.SemaphoreType.DMA((N_EXCH,)),
            pltpu.SemaphoreType.DMA((N_EXCH,)),
        ],
    )(x, Win0, Wout0, Win1, Wout1, Win2, Wout2)


# device time: 63730 ns/iter; 2.1282x vs baseline; 2.1282x over previous
import jax
import jax.numpy as jnp
from jax import lax
from jax.experimental import pallas as pl
from jax.experimental.pallas import tpu as pltpu

N_DEV = 8
N_LAYERS = 3
K_ORDER = (6, 2, 5, 7, 1, 3, 4)


def kernel(x, Win0, Wout0, Win1, Wout1, Win2, Wout2):
    b, d_shard = x.shape
    h_dim = Win0.shape[1]
    chunk = b // N_DEV

    def body(x_ref, win0_ref, wout0_ref, win1_ref, wout1_ref, win2_ref,
             wout2_ref, out_ref, acc_ref, hown_ref, hfull_ref, rs_buf,
             rs_send, rs_recv, ag_send, ag_recv, loc_sems):
        my = lax.axis_index("i")
        wins = (win0_ref, win1_ref, win2_ref)
        wouts = (wout0_ref, wout1_ref, wout2_ref)

        out_ref[...] = x_ref[...]

        prev_rs = []
        prev_ag = []
        for l in range(N_LAYERS):
            for r in prev_rs:
                r.wait_send()
            acc_ref[...] = jnp.dot(
                out_ref[...], wins[l][...],
                preferred_element_type=jnp.float32,
            ).reshape(N_DEV, chunk, h_dim)

            rs_rdmas = []
            for k in K_ORDER:
                d = my ^ k
                rdma = pltpu.make_async_remote_copy(
                    src_ref=acc_ref.at[d],
                    dst_ref=rs_buf.at[k],
                    send_sem=rs_send.at[k],
                    recv_sem=rs_recv.at[k],
                    device_id=(d,),
                    device_id_type=pl.DeviceIdType.MESH,
                )
                rdma.start()
                rs_rdmas.append(rdma)
            prev_rs = rs_rdmas

            own = pltpu.make_async_copy(
                acc_ref.at[my], rs_buf.at[0], loc_sems.at[0],
            )
            own.start()
            own.wait()

            for r in rs_rdmas:
                r.wait_recv()
            h = rs_buf[0]
            for k in range(1, N_DEV):
                h = h + rs_buf[k]
            h = jnp.maximum(h, 0.0)

            for r in prev_ag:
                r.wait_send()
            hown_ref[...] = h

            ag_rdmas = []
            for k in K_ORDER:
                d = my ^ k
                rdma = pltpu.make_async_remote_copy(
                    src_ref=hown_ref,
                    dst_ref=hfull_ref.at[my],
                    send_sem=ag_send.at[k],
                    recv_sem=ag_recv.at[k],
                    device_id=(d,),
                    device_id_type=pl.DeviceIdType.MESH,
                )
                rdma.start()
                ag_rdmas.append(rdma)
            prev_ag = ag_rdmas

            own = pltpu.make_async_copy(
                hown_ref, hfull_ref.at[my], loc_sems.at[1],
            )
            own.start()
            own.wait()
            for r in ag_rdmas:
                r.wait_recv()

            out_ref[...] = jnp.dot(
                hfull_ref[...].reshape(b, h_dim), wouts[l][...],
                preferred_element_type=jnp.float32,
            )

        for r in prev_rs:
            r.wait_send()
        for r in prev_ag:
            r.wait_send()

    return pl.pallas_call(
        body,
        out_shape=jax.ShapeDtypeStruct((b, d_shard), jnp.float32),
        in_specs=[pl.BlockSpec(memory_space=pltpu.VMEM)] * 7,
        out_specs=pl.BlockSpec(memory_space=pltpu.VMEM),
        scratch_shapes=[
            pltpu.VMEM((N_DEV, chunk, h_dim), jnp.float32),
            pltpu.VMEM((chunk, h_dim), jnp.float32),
            pltpu.VMEM((N_DEV, chunk, h_dim), jnp.float32),
            pltpu.VMEM((N_DEV, chunk, h_dim), jnp.float32),
            pltpu.SemaphoreType.DMA((N_DEV,)),
            pltpu.SemaphoreType.DMA((N_DEV,)),
            pltpu.SemaphoreType.DMA((N_DEV,)),
            pltpu.SemaphoreType.DMA((N_DEV,)),
            pltpu.SemaphoreType.DMA((2,)),
        ],
    )(x, Win0, Wout0, Win1, Wout1, Win2, Wout2)
